# Initial kernel scaffold; baseline (speedup 1.0000x reference)
import jax
import jax.numpy as jnp
from jax.experimental import pallas as pl


def kernel(index, emb, pre_fix, W, b):
    raise NotImplementedError("write your pallas kernel here")


# speedup vs baseline: 1.8258x; 1.8258x over previous
import functools

import jax
import jax.numpy as jnp
from jax import lax
from jax.experimental import pallas as pl
from jax.experimental.pallas import tpu as pltpu
from jax.experimental.pallas import tpu_sc as plsc

L = 16
NC = 2
NS = 16
NW = NC * NS
B = 16384
D = 144
V = 180
VP = 192
ROWS = B // NW
BLK = 2048


def _tc_body(emb_ref, pre_ref, w1_ref, w2_ref, b_ref, y_ref, s_ref):
    w2 = w2_ref[...]
    y_ref[...] = jnp.sum(emb_ref[...] * w2, axis=1) + b_ref[0]
    w1 = w1_ref[...]
    s_ref[...] = jnp.sum(pre_ref[...] * w1, axis=1)


def _tc_dense(emb, pre_pad, w1, w2, b):
    return pl.pallas_call(
        _tc_body,
        grid=(B // BLK,),
        in_specs=[
            pl.BlockSpec((BLK, D), lambda i: (i, 0)),
            pl.BlockSpec((VP, D), lambda i: (0, 0)),
            pl.BlockSpec((1, D), lambda i: (0, 0)),
            pl.BlockSpec((1, D), lambda i: (0, 0)),
            pl.BlockSpec((1,), lambda i: (0,)),
        ],
        out_specs=[
            pl.BlockSpec((BLK,), lambda i: (i,)),
            pl.BlockSpec((VP,), lambda i: (0,)),
        ],
        out_shape=[
            jax.ShapeDtypeStruct((B,), jnp.float32),
            jax.ShapeDtypeStruct((VP,), jnp.float32),
        ],
    )(emb, pre_pad, w1, w2, b)


def _sc_body(idx_hbm, y_hbm, s_hbm, out_hbm, idx_v, y_v, s_v, out_v):
    cid = lax.axis_index("c")
    sid = lax.axis_index("s")
    wid = cid * NS + sid
    base = wid * ROWS

    pltpu.sync_copy(idx_hbm.at[pl.ds(base, ROWS)], idx_v)
    pltpu.sync_copy(y_hbm.at[pl.ds(base, ROWS)], y_v)
    pltpu.sync_copy(s_hbm, s_v)

    chunks = [s_v[pl.ds(c * L, L)] for c in range(VP // L)]
    dnums = lax.GatherDimensionNumbers(
        offset_dims=(), collapsed_slice_dims=(0,), start_index_map=(0,))

    def vreg_gather(vals, lo):
        return lax.gather(
            vals, lo[:, None], dnums, (1,),
            mode=lax.GatherScatterMode.PROMISE_IN_BOUNDS)

    for g in range(ROWS // L):
        gi = idx_v[pl.ds(g * L, L)]
        hi = gi >> 4
        lo = gi & 15
        sv = jnp.zeros((L,), jnp.float32)
        for c in range(VP // L):
            sv = jnp.where(hi == c, vreg_gather(chunks[c], lo), sv)
        out_v[pl.ds(g * L, L)] = y_v[pl.ds(g * L, L)] + sv

    pltpu.sync_copy(out_v, out_hbm.at[pl.ds(base, ROWS)])


@functools.partial(
    pl.kernel,
    mesh=plsc.VectorSubcoreMesh(core_axis_name="c", subcore_axis_name="s"),
    out_type=jax.ShapeDtypeStruct((B,), jnp.float32),
    scratch_types=[
        pltpu.VMEM((ROWS,), jnp.int32),
        pltpu.VMEM((ROWS,), jnp.float32),
        pltpu.VMEM((VP,), jnp.float32),
        pltpu.VMEM((ROWS,), jnp.float32),
    ],
)
def _sc_gather_add(idx_hbm, y_hbm, s_hbm, out_hbm, *scratch):
    _sc_body(idx_hbm, y_hbm, s_hbm, out_hbm, *scratch)


def kernel(index, emb, pre_fix, W, b):
    w1 = W[:, :D]
    w2 = W[:, D:]
    pre_pad = jnp.pad(pre_fix, ((0, VP - V), (0, 0)))
    y, s = _tc_dense(emb, pre_pad, w1, w2, b)
    out = _sc_gather_add(index.astype(jnp.int32), y, s)
    return out.reshape(B, 1)
